# Initial kernel scaffold; baseline (speedup 1.0000x reference)
import jax
import jax.numpy as jnp
from jax.experimental import pallas as pl


def kernel(x, edge_index, W1, b1, W2, b2, Wout, bout):
    raise NotImplementedError("write your pallas kernel here")


# speedup vs baseline: 14.1274x; 14.1274x over previous
import functools

import jax
import jax.numpy as jnp
from jax import lax
from jax.experimental import pallas as pl
from jax.experimental.pallas import tpu as pltpu
from jax.experimental.pallas import tpu_sc as plsc

N = 100000
E = 1600000
ROWS = E // 128
NC, NS = 2, 16
RPC = ROWS // NC
R_TC = 2000
GRID = N // R_TC

def _zero_fill(zbuf, rows):
    def body(i, _):
        zbuf[i, :] = jnp.zeros((16,), jnp.float32)
        return 0
    lax.fori_loop(0, rows, body, 0)


@functools.cache
def _sc_deg_kernel():
    mesh = plsc.VectorSubcoreMesh(core_axis_name="c", subcore_axis_name="s")
    return pl.kernel(
        _sc_deg,
        out_type=jax.ShapeDtypeStruct((NC, N), jnp.float32),
        mesh=mesh,
        compiler_params=pltpu.CompilerParams(use_tc_tiling_on_sc=False),
        scratch_types=[
            pltpu.VMEM((1, 128), jnp.int32),
            pltpu.VMEM((128,), jnp.float32),
            pltpu.VMEM((2000,), jnp.float32),
            pltpu.VMEM_SHARED((N,), jnp.float32),
        ],
    )


def _sc_deg(dst_hbm, out_hbm, dbuf, ones, zbuf, acc):
    c = lax.axis_index("c")
    s = lax.axis_index("s")

    def ob(i, _):
        ones[pl.ds(i * 16, 16)] = jnp.full((16,), 1.0, jnp.float32)
        return 0
    lax.fori_loop(0, 8, ob, 0)

    def zb(i, _):
        zbuf[pl.ds(i * 16, 16)] = jnp.zeros((16,), jnp.float32)
        return 0
    lax.fori_loop(0, 125, zb, 0)

    zlo = (s * 50) // 16
    zhi = ((s + 1) * 50) // 16

    def zc(i, _):
        pltpu.sync_copy(zbuf, acc.at[pl.ds(i * 2000, 2000)])
        return 0
    lax.fori_loop(zlo, zhi, zc, 0)
    plsc.subcore_barrier()

    lo = c * RPC + (s * RPC) // 16
    hi = c * RPC + ((s + 1) * RPC) // 16

    def body(r, _):
        pltpu.sync_copy(dst_hbm.at[r], dbuf.at[0])
        pltpu.sync_copy(ones, acc.at[dbuf.at[0]], add=True)
        return 0
    lax.fori_loop(lo, hi, body, 0)
    plsc.subcore_barrier()

    @pl.when(s == 0)
    def _():
        pltpu.sync_copy(acc, out_hbm.at[c])


@functools.cache
def _sc_agg1_kernel():
    mesh = plsc.VectorSubcoreMesh(core_axis_name="c", subcore_axis_name="s")
    return pl.kernel(
        _sc_agg1,
        out_type=jax.ShapeDtypeStruct((NC, N, 16), jnp.float32),
        mesh=mesh,
        compiler_params=pltpu.CompilerParams(use_tc_tiling_on_sc=False),
        scratch_types=[
            pltpu.VMEM((1, 128), jnp.int32),
            pltpu.VMEM((1, 128), jnp.int32),
            pltpu.VMEM((128, 16), jnp.float32),
            pltpu.VMEM((125, 16), jnp.float32),
            pltpu.VMEM_SHARED((N, 16), jnp.float32),
            pltpu.SemaphoreType.DMA,
        ],
    )


def _sc_agg1(src_hbm, dst_hbm, y1_hbm, out_hbm, sbuf, dbuf, rbuf, zbuf, acc, gsem):
    c = lax.axis_index("c")
    s = lax.axis_index("s")
    _zero_fill(zbuf, 125)

    def zc(i, _):
        pltpu.sync_copy(zbuf, acc.at[pl.ds(i * 125, 125)])
        return 0
    lax.fori_loop(s * 50, (s + 1) * 50, zc, 0)
    plsc.subcore_barrier()

    lo = c * RPC + (s * RPC) // 16
    hi = c * RPC + ((s + 1) * RPC) // 16

    def body(r, _):
        pltpu.sync_copy(src_hbm.at[r], sbuf.at[0])
        pltpu.sync_copy(dst_hbm.at[r], dbuf.at[0])
        pltpu.async_copy(y1_hbm.at[sbuf.at[0]], rbuf, gsem).wait()
        pltpu.sync_copy(rbuf, acc.at[dbuf.at[0]], add=True)
        return 0
    lax.fori_loop(lo, hi, body, 0)
    plsc.subcore_barrier()

    def wb(i, _):
        off = pl.multiple_of(i * 2000, 8)
        pltpu.sync_copy(acc.at[pl.ds(off, 2000)], out_hbm.at[c, pl.ds(off, 2000)])
        return 0
    lax.fori_loop((s * 50) // 16, ((s + 1) * 50) // 16, wb, 0)


@functools.cache
def _sc_agg2_kernel():
    mesh = plsc.VectorSubcoreMesh(core_axis_name="c", subcore_axis_name="s")
    return pl.kernel(
        _sc_agg2,
        out_type=jax.ShapeDtypeStruct((NC, N, 16), jnp.float32),
        mesh=mesh,
        compiler_params=pltpu.CompilerParams(use_tc_tiling_on_sc=False),
        scratch_types=[
            pltpu.VMEM((1, 128), jnp.int32),
            pltpu.VMEM((1, 128), jnp.int32),
            pltpu.VMEM((128, 16), jnp.float32),
            pltpu.VMEM((125, 16), jnp.float32),
            pltpu.VMEM_SHARED((N, 16), jnp.float32),
            pltpu.SemaphoreType.DMA,
        ],
    )


def _sc_agg2(src_hbm, dst_hbm, y2a_hbm, y2b_hbm, out_hbm, sbuf, dbuf, rbuf, zbuf, acc, gsem):
    c = lax.axis_index("c")
    s = lax.axis_index("s")
    _zero_fill(zbuf, 125)

    def zc(i, _):
        pltpu.sync_copy(zbuf, acc.at[pl.ds(i * 125, 125)])
        return 0
    lax.fori_loop(s * 50, (s + 1) * 50, zc, 0)
    plsc.subcore_barrier()

    lo = (s * ROWS) // 16
    hi = ((s + 1) * ROWS) // 16

    def make_body(table):
        def body(r, _):
            pltpu.sync_copy(src_hbm.at[r], sbuf.at[0])
            pltpu.sync_copy(dst_hbm.at[r], dbuf.at[0])
            pltpu.async_copy(table.at[sbuf.at[0]], rbuf, gsem).wait()
            pltpu.sync_copy(rbuf, acc.at[dbuf.at[0]], add=True)
            return 0
        return body

    @pl.when(c == 0)
    def _():
        lax.fori_loop(lo, hi, make_body(y2a_hbm), 0)

    @pl.when(c == 1)
    def _():
        lax.fori_loop(lo, hi, make_body(y2b_hbm), 0)

    plsc.subcore_barrier()

    def wb(i, _):
        off = pl.multiple_of(i * 2000, 8)
        pltpu.sync_copy(acc.at[pl.ds(off, 2000)], out_hbm.at[c, pl.ds(off, 2000)])
        return 0
    lax.fori_loop((s * 50) // 16, ((s + 1) * 50) // 16, wb, 0)


def _tc1_body(degp_ref, x_ref, dinv_ref, y1_ref):
    deg = degp_ref[0] + degp_ref[1] + 1.0
    dinv = lax.rsqrt(deg)
    dinv_ref[...] = dinv
    y1_ref[...] = x_ref[...] * dinv


def _tc1(degp, xpad):
    return pl.pallas_call(
        _tc1_body,
        grid=(GRID,),
        in_specs=[
            pl.BlockSpec((2, R_TC, 1), lambda i: (0, i, 0)),
            pl.BlockSpec((R_TC, 16), lambda i: (i, 0)),
        ],
        out_specs=[
            pl.BlockSpec((R_TC, 1), lambda i: (i, 0)),
            pl.BlockSpec((R_TC, 16), lambda i: (i, 0)),
        ],
        out_shape=[
            jax.ShapeDtypeStruct((N, 1), jnp.float32),
            jax.ShapeDtypeStruct((N, 16), jnp.float32),
        ],
    )(degp, xpad)


def _tc2_body(aggp_ref, y1_ref, dinv_ref, w1_ref, b1_ref, w2_ref, y2a_ref, y2b_ref):
    dinv = dinv_ref[...]
    prop1 = (aggp_ref[0] + aggp_ref[1] + y1_ref[...]) * dinv
    h1 = jnp.maximum(
        jnp.dot(prop1, w1_ref[...], preferred_element_type=jnp.float32)
        + b1_ref[...],
        0.0,
    )
    z = jnp.dot(h1, w2_ref[...], preferred_element_type=jnp.float32)
    y2 = z * dinv
    y2a_ref[...] = y2[:, :16]
    y2b_ref[...] = y2[:, 16:]


def _tc2(agg1p, y1, dinv, w1p, b1r, w2):
    return pl.pallas_call(
        _tc2_body,
        grid=(GRID,),
        in_specs=[
            pl.BlockSpec((2, R_TC, 16), lambda i: (0, i, 0)),
            pl.BlockSpec((R_TC, 16), lambda i: (i, 0)),
            pl.BlockSpec((R_TC, 1), lambda i: (i, 0)),
            pl.BlockSpec((16, 64), lambda i: (0, 0)),
            pl.BlockSpec((1, 64), lambda i: (0, 0)),
            pl.BlockSpec((64, 32), lambda i: (0, 0)),
        ],
        out_specs=[
            pl.BlockSpec((R_TC, 16), lambda i: (i, 0)),
            pl.BlockSpec((R_TC, 16), lambda i: (i, 0)),
        ],
        out_shape=[
            jax.ShapeDtypeStruct((N, 16), jnp.float32),
            jax.ShapeDtypeStruct((N, 16), jnp.float32),
        ],
    )(agg1p, y1, dinv, w1p, b1r, w2)


def _tc3_body(agg2p_ref, y2a_ref, y2b_ref, dinv_ref, b2_ref, wout_ref, bout_ref,
              out_ref, acc_ref):
    i = pl.program_id(0)
    dinv = dinv_ref[...]
    h2a = jnp.maximum((agg2p_ref[0] + y2a_ref[...]) * dinv + b2_ref[:, :16], 0.0)
    h2b = jnp.maximum((agg2p_ref[1] + y2b_ref[...]) * dinv + b2_ref[:, 16:], 0.0)
    ps = jnp.concatenate(
        [jnp.sum(h2a, axis=0, keepdims=True), jnp.sum(h2b, axis=0, keepdims=True)],
        axis=1,
    )

    @pl.when(i == 0)
    def _():
        acc_ref[...] = ps

    @pl.when(i > 0)
    def _():
        acc_ref[...] = acc_ref[...] + ps

    @pl.when(i == GRID - 1)
    def _():
        g = acc_ref[...] * (1.0 / N)
        t = jnp.dot(g, wout_ref[...], preferred_element_type=jnp.float32) \
            + bout_ref[...]
        out_ref[...] = 1.0 / (1.0 + jnp.exp(-t))


def _tc3(agg2p, y2a, y2b, dinv, b2r, wout, boutr):
    return pl.pallas_call(
        _tc3_body,
        grid=(GRID,),
        in_specs=[
            pl.BlockSpec((2, R_TC, 16), lambda i: (0, i, 0)),
            pl.BlockSpec((R_TC, 16), lambda i: (i, 0)),
            pl.BlockSpec((R_TC, 16), lambda i: (i, 0)),
            pl.BlockSpec((R_TC, 1), lambda i: (i, 0)),
            pl.BlockSpec((1, 32), lambda i: (0, 0)),
            pl.BlockSpec((32, 1), lambda i: (0, 0)),
            pl.BlockSpec((1, 1), lambda i: (0, 0)),
        ],
        out_specs=pl.BlockSpec((1, 1), lambda i: (0, 0)),
        out_shape=jax.ShapeDtypeStruct((1, 1), jnp.float32),
        scratch_shapes=[pltpu.VMEM((1, 32), jnp.float32)],
    )(agg2p, y2a, y2b, dinv, b2r, wout, boutr)


@jax.jit
def kernel(x, edge_index, W1, b1, W2, b2, Wout, bout):
    src2d = edge_index[0].reshape(ROWS, 128)
    dst2d = edge_index[1].reshape(ROWS, 128)
    xpad = jnp.pad(x, ((0, 0), (0, 16 - x.shape[1])))
    w1p = jnp.pad(W1, ((0, 16 - W1.shape[0]), (0, 0)))

    degp = _sc_deg_kernel()(dst2d)
    dinv, y1 = _tc1(degp.reshape(NC, N, 1), xpad)
    agg1p = _sc_agg1_kernel()(src2d, dst2d, y1)
    y2a, y2b = _tc2(agg1p, y1, dinv, w1p, b1.reshape(1, 64), W2)
    agg2p = _sc_agg2_kernel()(src2d, dst2d, y2a, y2b)
    out = _tc3(agg2p, y2a, y2b, dinv, b2.reshape(1, 32), Wout, bout.reshape(1, 1))
    return out.reshape(1)


# speedup vs baseline: 47.1695x; 3.3389x over previous
import functools

import jax
import jax.numpy as jnp
from jax import lax
from jax.experimental import pallas as pl
from jax.experimental.pallas import tpu as pltpu
from jax.experimental.pallas import tpu_sc as plsc

N = 100000
E = 1600000
NC, NS = 2, 16
K = 14
KA = 4
ROWS_PAD = 12544
EPAD = ROWS_PAD * 128
RPC = ROWS_PAD // NC
NA = N + 8
R_TC = 2000
GRID = N // R_TC

_SC_PARAMS = pltpu.CompilerParams(use_tc_tiling_on_sc=False)


def _zero_fill(zbuf, rows):
    def body(i, _):
        zbuf[i, :] = jnp.zeros((16,), jnp.float32)
        return 0
    lax.fori_loop(0, rows, body, 0)


def _zero_acc(zbuf, acc, s, zsem):
    def zc(i, _):
        pltpu.async_copy(zbuf, acc.at[pl.ds(i * 125, 125)], zsem)
        return 0
    lax.fori_loop(s * 50, (s + 1) * 50, zc, 0)

    def zw(i, _):
        pltpu.make_async_copy(zbuf, acc.at[pl.ds(0, 125)], zsem).wait()
        return 0
    lax.fori_loop(0, 50, zw, 0)


def _writeback(acc, out_hbm, c, s):
    def wb(i, _):
        off = pl.multiple_of(i * 2000, 8)
        pltpu.sync_copy(acc.at[pl.ds(off, 2000)], out_hbm.at[c, pl.ds(off, 2000)])
        return 0
    lax.fori_loop((s * 50) // 16, ((s + 1) * 50) // 16, wb, 0)


@functools.cache
def _sc_deg_kernel():
    mesh = plsc.VectorSubcoreMesh(core_axis_name="c", subcore_axis_name="s")
    return pl.kernel(
        _sc_deg,
        out_type=jax.ShapeDtypeStruct((NC, N), jnp.float32),
        mesh=mesh,
        compiler_params=_SC_PARAMS,
        scratch_types=[
            pltpu.VMEM((2, K, 128), jnp.int32),
            pltpu.VMEM((128,), jnp.float32),
            pltpu.VMEM((2000,), jnp.float32),
            pltpu.VMEM_SHARED((NA,), jnp.float32),
            pltpu.SemaphoreType.DMA,
            pltpu.SemaphoreType.DMA,
            pltpu.SemaphoreType.DMA,
            pltpu.SemaphoreType.DMA,
        ],
    )


def _sc_deg(dst_hbm, out_hbm, dbuf, ones, zbuf, acc, isem, ssem0, ssem1, zsem):
    c = lax.axis_index("c")
    s = lax.axis_index("s")
    ssems = (ssem0, ssem1)

    def ob(i, _):
        ones[pl.ds(i * 16, 16)] = jnp.full((16,), 1.0, jnp.float32)
        return 0
    lax.fori_loop(0, 8, ob, 0)

    def zb(i, _):
        zbuf[pl.ds(i * 16, 16)] = jnp.zeros((16,), jnp.float32)
        return 0
    lax.fori_loop(0, 125, zb, 0)

    def zc(i, _):
        pltpu.async_copy(zbuf, acc.at[pl.ds(i * 2000, 2000)], zsem)
        return 0
    lax.fori_loop((s * 50) // 16, ((s + 1) * 50) // 16, zc, 0)

    def zw(i, _):
        pltpu.make_async_copy(zbuf, acc.at[pl.ds(0, 2000)], zsem).wait()
        return 0
    lax.fori_loop((s * 50) // 16, ((s + 1) * 50) // 16, zw, 0)
    plsc.subcore_barrier()

    row0 = c * RPC + s * (RPC // NS)
    n_chunks = RPC // NS // K

    def step(g, slot):
        pltpu.make_async_copy(dst_hbm.at[pl.ds(0, K)], dbuf.at[slot], isem).wait()

        @pl.when(g + 1 < n_chunks)
        def _():
            r = row0 + (g + 1) * K
            pltpu.async_copy(dst_hbm.at[pl.ds(r, K)], dbuf.at[1 - slot], isem)

        @pl.when(g >= 2)
        def _():
            for j in range(K):
                pltpu.make_async_copy(ones, acc.at[dbuf.at[slot, j]], ssems[slot]).wait()

        for j in range(K):
            pltpu.async_copy(ones, acc.at[dbuf.at[slot, j]], ssems[slot], add=True)

    pltpu.async_copy(dst_hbm.at[pl.ds(row0, K)], dbuf.at[0], isem)

    def loop(g2, _):
        step(2 * g2, 0)
        step(2 * g2 + 1, 1)
        return 0
    lax.fori_loop(0, n_chunks // 2, loop, 0)

    for slot in (0, 1):
        for j in range(K):
            pltpu.make_async_copy(ones, acc.at[dbuf.at[slot, j]], ssems[slot]).wait()
    plsc.subcore_barrier()

    @pl.when(s == 0)
    def _():
        pltpu.sync_copy(acc.at[pl.ds(0, N)], out_hbm.at[c])


def _agg_pipeline(src_hbm, dst_hbm, table, acc, sbuf, dbuf, rbuf,
                  isem, gsem, ssems, row0, n_chunks):
    def step(g, slot):
        pltpu.make_async_copy(src_hbm.at[pl.ds(0, KA)], sbuf.at[slot], isem).wait()
        pltpu.make_async_copy(dst_hbm.at[pl.ds(0, KA)], dbuf.at[slot], isem).wait()

        @pl.when(g + 1 < n_chunks)
        def _():
            r = row0 + (g + 1) * KA
            pltpu.async_copy(src_hbm.at[pl.ds(r, KA)], sbuf.at[1 - slot], isem)
            pltpu.async_copy(dst_hbm.at[pl.ds(r, KA)], dbuf.at[1 - slot], isem)

        @pl.when(g >= 2)
        def _():
            for j in range(KA):
                pltpu.make_async_copy(
                    rbuf.at[slot, j], acc.at[dbuf.at[slot, j]], ssems[slot]
                ).wait()

        descs = [
            pltpu.async_copy(table.at[sbuf.at[slot, j]], rbuf.at[slot, j], gsem)
            for j in range(KA)
        ]
        for d in descs:
            d.wait()
        for j in range(KA):
            pltpu.async_copy(
                rbuf.at[slot, j], acc.at[dbuf.at[slot, j]], ssems[slot], add=True
            )

    pltpu.async_copy(src_hbm.at[pl.ds(row0, KA)], sbuf.at[0], isem)
    pltpu.async_copy(dst_hbm.at[pl.ds(row0, KA)], dbuf.at[0], isem)

    def loop(g2, _):
        step(2 * g2, 0)
        step(2 * g2 + 1, 1)
        return 0
    lax.fori_loop(0, n_chunks // 2, loop, 0)

    for slot in (0, 1):
        for j in range(KA):
            pltpu.make_async_copy(
                rbuf.at[slot, j], acc.at[dbuf.at[slot, j]], ssems[slot]
            ).wait()


_AGG_SCRATCH = [
    pltpu.VMEM((2, KA, 128), jnp.int32),
    pltpu.VMEM((2, KA, 128), jnp.int32),
    pltpu.VMEM((2, KA, 128, 16), jnp.float32),
    pltpu.VMEM((125, 16), jnp.float32),
    pltpu.VMEM_SHARED((NA, 16), jnp.float32),
    pltpu.SemaphoreType.DMA,
    pltpu.SemaphoreType.DMA,
    pltpu.SemaphoreType.DMA,
    pltpu.SemaphoreType.DMA,
    pltpu.SemaphoreType.DMA,
]


@functools.cache
def _sc_agg1_kernel():
    mesh = plsc.VectorSubcoreMesh(core_axis_name="c", subcore_axis_name="s")
    return pl.kernel(
        _sc_agg1,
        out_type=jax.ShapeDtypeStruct((NC, N, 16), jnp.float32),
        mesh=mesh,
        compiler_params=_SC_PARAMS,
        scratch_types=list(_AGG_SCRATCH),
    )


def _sc_agg1(src_hbm, dst_hbm, y1_hbm, out_hbm, sbuf, dbuf, rbuf, zbuf, acc,
             isem, gsem, ssem0, ssem1, zsem):
    c = lax.axis_index("c")
    s = lax.axis_index("s")
    _zero_fill(zbuf, 125)
    _zero_acc(zbuf, acc, s, zsem)
    plsc.subcore_barrier()

    row0 = c * RPC + s * (RPC // NS)
    _agg_pipeline(src_hbm, dst_hbm, y1_hbm, acc, sbuf, dbuf, rbuf,
                  isem, gsem, (ssem0, ssem1), row0, (RPC // NS) // KA)
    plsc.subcore_barrier()
    _writeback(acc, out_hbm, c, s)


@functools.cache
def _sc_agg2_kernel():
    mesh = plsc.VectorSubcoreMesh(core_axis_name="c", subcore_axis_name="s")
    return pl.kernel(
        _sc_agg2,
        out_type=jax.ShapeDtypeStruct((NC, N, 16), jnp.float32),
        mesh=mesh,
        compiler_params=_SC_PARAMS,
        scratch_types=list(_AGG_SCRATCH),
    )


def _sc_agg2(src_hbm, dst_hbm, y2a_hbm, y2b_hbm, out_hbm, sbuf, dbuf, rbuf,
             zbuf, acc, isem, gsem, ssem0, ssem1, zsem):
    c = lax.axis_index("c")
    s = lax.axis_index("s")
    _zero_fill(zbuf, 125)
    _zero_acc(zbuf, acc, s, zsem)
    plsc.subcore_barrier()

    row0 = s * (ROWS_PAD // NS)
    n_chunks = ROWS_PAD // NS // KA

    @pl.when(c == 0)
    def _():
        _agg_pipeline(src_hbm, dst_hbm, y2a_hbm, acc, sbuf, dbuf, rbuf,
                      isem, gsem, (ssem0, ssem1), row0, n_chunks)

    @pl.when(c == 1)
    def _():
        _agg_pipeline(src_hbm, dst_hbm, y2b_hbm, acc, sbuf, dbuf, rbuf,
                      isem, gsem, (ssem0, ssem1), row0, n_chunks)

    plsc.subcore_barrier()
    _writeback(acc, out_hbm, c, s)


def _tc1_body(degp_ref, x_ref, dinv_ref, y1_ref):
    deg = degp_ref[0] + degp_ref[1] + 1.0
    dinv = lax.rsqrt(deg)
    dinv_ref[...] = dinv
    y1_ref[...] = x_ref[...] * dinv


def _tc1(degp, xpad):
    return pl.pallas_call(
        _tc1_body,
        grid=(GRID,),
        in_specs=[
            pl.BlockSpec((2, R_TC, 1), lambda i: (0, i, 0)),
            pl.BlockSpec((R_TC, 16), lambda i: (i, 0)),
        ],
        out_specs=[
            pl.BlockSpec((R_TC, 1), lambda i: (i, 0)),
            pl.BlockSpec((R_TC, 16), lambda i: (i, 0)),
        ],
        out_shape=[
            jax.ShapeDtypeStruct((N, 1), jnp.float32),
            jax.ShapeDtypeStruct((N, 16), jnp.float32),
        ],
    )(degp, xpad)


def _tc2_body(aggp_ref, y1_ref, dinv_ref, w1_ref, b1_ref, w2_ref, y2a_ref, y2b_ref):
    dinv = dinv_ref[...]
    prop1 = (aggp_ref[0] + aggp_ref[1] + y1_ref[...]) * dinv
    h1 = jnp.maximum(
        jnp.dot(prop1, w1_ref[...], preferred_element_type=jnp.float32)
        + b1_ref[...],
        0.0,
    )
    z = jnp.dot(h1, w2_ref[...], preferred_element_type=jnp.float32)
    y2 = z * dinv
    y2a_ref[...] = y2[:, :16]
    y2b_ref[...] = y2[:, 16:]


def _tc2(agg1p, y1, dinv, w1p, b1r, w2):
    return pl.pallas_call(
        _tc2_body,
        grid=(GRID,),
        in_specs=[
            pl.BlockSpec((2, R_TC, 16), lambda i: (0, i, 0)),
            pl.BlockSpec((R_TC, 16), lambda i: (i, 0)),
            pl.BlockSpec((R_TC, 1), lambda i: (i, 0)),
            pl.BlockSpec((16, 64), lambda i: (0, 0)),
            pl.BlockSpec((1, 64), lambda i: (0, 0)),
            pl.BlockSpec((64, 32), lambda i: (0, 0)),
        ],
        out_specs=[
            pl.BlockSpec((R_TC, 16), lambda i: (i, 0)),
            pl.BlockSpec((R_TC, 16), lambda i: (i, 0)),
        ],
        out_shape=[
            jax.ShapeDtypeStruct((N, 16), jnp.float32),
            jax.ShapeDtypeStruct((N, 16), jnp.float32),
        ],
    )(agg1p, y1, dinv, w1p, b1r, w2)


def _tc3_body(agg2p_ref, y2a_ref, y2b_ref, dinv_ref, b2_ref, wout_ref, bout_ref,
              out_ref, acc_ref):
    i = pl.program_id(0)
    dinv = dinv_ref[...]
    h2a = jnp.maximum((agg2p_ref[0] + y2a_ref[...]) * dinv + b2_ref[:, :16], 0.0)
    h2b = jnp.maximum((agg2p_ref[1] + y2b_ref[...]) * dinv + b2_ref[:, 16:], 0.0)
    ps = jnp.concatenate(
        [jnp.sum(h2a, axis=0, keepdims=True), jnp.sum(h2b, axis=0, keepdims=True)],
        axis=1,
    )

    @pl.when(i == 0)
    def _():
        acc_ref[...] = ps

    @pl.when(i > 0)
    def _():
        acc_ref[...] = acc_ref[...] + ps

    @pl.when(i == GRID - 1)
    def _():
        g = acc_ref[...] * (1.0 / N)
        t = jnp.dot(g, wout_ref[...], preferred_element_type=jnp.float32) \
            + bout_ref[...]
        out_ref[...] = 1.0 / (1.0 + jnp.exp(-t))


def _tc3(agg2p, y2a, y2b, dinv, b2r, wout, boutr):
    return pl.pallas_call(
        _tc3_body,
        grid=(GRID,),
        in_specs=[
            pl.BlockSpec((2, R_TC, 16), lambda i: (0, i, 0)),
            pl.BlockSpec((R_TC, 16), lambda i: (i, 0)),
            pl.BlockSpec((R_TC, 16), lambda i: (i, 0)),
            pl.BlockSpec((R_TC, 1), lambda i: (i, 0)),
            pl.BlockSpec((1, 32), lambda i: (0, 0)),
            pl.BlockSpec((32, 1), lambda i: (0, 0)),
            pl.BlockSpec((1, 1), lambda i: (0, 0)),
        ],
        out_specs=pl.BlockSpec((1, 1), lambda i: (0, 0)),
        out_shape=jax.ShapeDtypeStruct((1, 1), jnp.float32),
        scratch_shapes=[pltpu.VMEM((1, 32), jnp.float32)],
    )(agg2p, y2a, y2b, dinv, b2r, wout, boutr)


@jax.jit
def kernel(x, edge_index, W1, b1, W2, b2, Wout, bout):
    pad = EPAD - E
    src2d = jnp.concatenate(
        [edge_index[0], jnp.zeros((pad,), jnp.int32)]).reshape(ROWS_PAD, 128)
    dst2d = jnp.concatenate(
        [edge_index[1], jnp.full((pad,), N, jnp.int32)]).reshape(ROWS_PAD, 128)
    xpad = jnp.pad(x, ((0, 0), (0, 16 - x.shape[1])))
    w1p = jnp.pad(W1, ((0, 16 - W1.shape[0]), (0, 0)))

    degp = _sc_deg_kernel()(dst2d)
    dinv, y1 = _tc1(degp.reshape(NC, N, 1), xpad)
    agg1p = _sc_agg1_kernel()(src2d, dst2d, y1)
    y2a, y2b = _tc2(agg1p, y1, dinv, w1p, b1.reshape(1, 64), W2)
    agg2p = _sc_agg2_kernel()(src2d, dst2d, y2a, y2b)
    out = _tc3(agg2p, y2a, y2b, dinv, b2.reshape(1, 32), Wout, bout.reshape(1, 1))
    return out.reshape(1)
